# baseline (speedup 1.0000x reference)
import jax
import jax.numpy as jnp
from jax.experimental import pallas as pl
from jax.experimental.pallas import tpu as pltpu

_EMBED = 4096
_E = 64
_K = 8
_ROWS = 512


def _router_block(x_ref, w_ref, b_ref, scores_ref, idx_ref):
    x = x_ref[...]
    w = w_ref[...]
    logits = jax.lax.dot_general(
        x, w, (((1,), (1,)), ((), ())), preferred_element_type=jnp.float32
    ) + b_ref[...]

    lt = logits.T
    rows = jax.lax.broadcasted_iota(jnp.int32, lt.shape, 0)
    vals = lt
    idxs = []
    m0 = None
    for k in range(_K):
        m = jnp.max(vals, axis=0, keepdims=True)
        if k == 0:
            m0 = m
        a = jnp.min(jnp.where(vals == m, rows, _E), axis=0, keepdims=True)
        idxs.append(a)
        vals = jnp.where(rows == a, -jnp.inf, vals)

    sel = vals == -jnp.inf
    e = jnp.where(sel, jnp.exp(lt - m0), 0.0)
    denom = jnp.sum(e, axis=0, keepdims=True)
    scores_ref[...] = (e / denom).T
    idx_ref[...] = jnp.concatenate(idxs, axis=0).T


def kernel(hidden_states, weight, bias):
    flat = hidden_states.reshape(-1, _EMBED)
    n_rows = flat.shape[0]
    grid = n_rows // _ROWS
    bias2d = bias.reshape(1, _E)

    scores, idx = pl.pallas_call(
        _router_block,
        grid=(grid,),
        in_specs=[
            pl.BlockSpec((_ROWS, _EMBED), lambda i: (i, 0)),
            pl.BlockSpec((_E, _EMBED), lambda i: (0, 0)),
            pl.BlockSpec((1, _E), lambda i: (0, 0)),
        ],
        out_specs=[
            pl.BlockSpec((_ROWS, _E), lambda i: (i, 0)),
            pl.BlockSpec((_ROWS, _K), lambda i: (i, 0)),
        ],
        out_shape=[
            jax.ShapeDtypeStruct((n_rows, _E), jnp.float32),
            jax.ShapeDtypeStruct((n_rows, _K), jnp.int32),
        ],
    )(flat, weight, bias2d)
    return (scores, idx)


# speedup vs baseline: 1.0499x; 1.0499x over previous
import jax
import jax.numpy as jnp
from jax.experimental import pallas as pl
from jax.experimental.pallas import tpu as pltpu

_EMBED = 4096
_E = 64
_K = 8
_ROWS = 1024


def _router_block(x_ref, w_ref, b_ref, scores_ref, idx_ref):
    x = x_ref[...]
    w = w_ref[...]
    logits = jax.lax.dot_general(
        x, w, (((1,), (1,)), ((), ())), preferred_element_type=jnp.float32
    ) + b_ref[...]

    lt = logits.T
    rows = jax.lax.broadcasted_iota(jnp.int32, lt.shape, 0)
    vals = lt
    idxs = []
    m0 = None
    for k in range(_K):
        m = jnp.max(vals, axis=0, keepdims=True)
        if k == 0:
            m0 = m
        a = jnp.min(jnp.where(vals == m, rows, _E), axis=0, keepdims=True)
        idxs.append(a)
        vals = jnp.where(rows == a, -jnp.inf, vals)

    sel = vals == -jnp.inf
    e = jnp.where(sel, jnp.exp(lt - m0), 0.0)
    denom = jnp.sum(e, axis=0, keepdims=True)
    scores_ref[...] = (e / denom).T
    idx_ref[...] = jnp.concatenate(idxs, axis=0).T


def kernel(hidden_states, weight, bias):
    flat = hidden_states.reshape(-1, _EMBED)
    n_rows = flat.shape[0]
    grid = n_rows // _ROWS
    bias2d = bias.reshape(1, _E)

    scores, idx = pl.pallas_call(
        _router_block,
        grid=(grid,),
        in_specs=[
            pl.BlockSpec((_ROWS, _EMBED), lambda i: (i, 0)),
            pl.BlockSpec((_E, _EMBED), lambda i: (0, 0)),
            pl.BlockSpec((1, _E), lambda i: (0, 0)),
        ],
        out_specs=[
            pl.BlockSpec((_ROWS, _E), lambda i: (i, 0)),
            pl.BlockSpec((_ROWS, _K), lambda i: (i, 0)),
        ],
        out_shape=[
            jax.ShapeDtypeStruct((n_rows, _E), jnp.float32),
            jax.ShapeDtypeStruct((n_rows, _K), jnp.int32),
        ],
    )(flat, weight, bias2d)
    return (scores, idx)
